# Initial kernel scaffold; baseline (speedup 1.0000x reference)
import jax
import jax.numpy as jnp
from jax.experimental import pallas as pl


def kernel(x, edge_index, score, Wl0, bl0, Wr0, br0, Wl1, bl1, Wr1, br1, Wp, bp):
    raise NotImplementedError("write your pallas kernel here")


# speedup vs baseline: 1.0154x; 1.0154x over previous
import functools

import jax
import jax.numpy as jnp
from jax import lax
from jax.experimental import pallas as pl
from jax.experimental.pallas import tpu as pltpu
from jax.experimental.pallas import tpu_sc as plsc

N = 10000
D = 128
E = 320000

NC = 2
NS = 16
NW = NC * NS

CH = 128
NCHUNK = -(-E // (NW * CH))
EPT = NCHUNK * CH
E_PAD = EPT * NW

NPAD = 10112
RPT = NPAD // NS


def _make_sc_aggregate(with_cnt: bool):
  mesh = plsc.VectorSubcoreMesh(core_axis_name="c", subcore_axis_name="s")

  out_type = [jax.ShapeDtypeStruct((NC, NPAD, D), jnp.float32)]
  if with_cnt:
    out_type.append(jax.ShapeDtypeStruct((NC, NPAD, 16), jnp.float32))

  scratch = [
      pltpu.VMEM_SHARED((NPAD, D), jnp.float32),
      pltpu.VMEM((CH,), jnp.int32),
      pltpu.VMEM((CH,), jnp.int32),
      pltpu.VMEM((CH,), jnp.float32),
      pltpu.VMEM((CH, D), jnp.float32),
      pltpu.SemaphoreType.DMA,
  ]
  if with_cnt:
    scratch += [
        pltpu.VMEM_SHARED((NPAD, 16), jnp.float32),
        pltpu.VMEM((CH, 16), jnp.float32),
    ]

  def body(feats, srcp, dstp, scorep, zrows, zcnt, ones, *rest):
    if with_cnt:
      acc_out, cnt_out, acc_s, src_v, dst_v, sc_v, rows_v, sem, cnt_s, ones_v = rest
    else:
      acc_out, acc_s, src_v, dst_v, sc_v, rows_v, sem = rest
    c = lax.axis_index("c")
    s = lax.axis_index("s")
    wid = s * NC + c

    pltpu.sync_copy(zrows, acc_s.at[pl.ds(s * RPT, RPT)])
    if with_cnt:
      pltpu.sync_copy(zcnt, cnt_s.at[pl.ds(s * RPT, RPT)])
      pltpu.sync_copy(ones, ones_v)
    plsc.subcore_barrier()

    base0 = wid * EPT

    def chunk(i, carry):
      b = base0 + i * CH
      pltpu.sync_copy(srcp.at[pl.ds(b, CH)], src_v)
      pltpu.sync_copy(dstp.at[pl.ds(b, CH)], dst_v)
      pltpu.sync_copy(scorep.at[pl.ds(b, CH)], sc_v)
      pltpu.async_copy(feats.at[src_v], rows_v, sem).wait()

      for g in range(CH // 16):
        sv16 = sc_v[pl.ds(g * 16, 16)]
        for j in range(16):
          k = g * 16 + j
          sv = jnp.full((16,), sv16[j], jnp.float32)
          for f in range(D // 16):
            rows_v[k, pl.ds(f * 16, 16)] = rows_v[k, pl.ds(f * 16, 16)] * sv

      return carry

    plsc.subcore_barrier()

    pltpu.sync_copy(acc_s.at[pl.ds(s * RPT, RPT)],
                    acc_out.at[c, pl.ds(s * RPT, RPT)])
    if with_cnt:
      pltpu.sync_copy(cnt_s.at[pl.ds(s * RPT, RPT)],
                      cnt_out.at[c, pl.ds(s * RPT, RPT)])

  return pl.kernel(body, out_type=tuple(out_type), mesh=mesh,
                   scratch_types=scratch)


_sc_agg_cnt = _make_sc_aggregate(True)
_sc_agg = _make_sc_aggregate(False)

_TB = 1000


def _tc_layer_body(x_ref, a_ref, c_ref, wl_ref, wr_ref, b_ref, o_ref):
  a = a_ref[0] + a_ref[1]
  cnt = c_ref[0, :, 0:1] + c_ref[1, :, 0:1]
  mean = a / jnp.maximum(cnt, 1.0)
  h = lax.dot_general(x_ref[...], wl_ref[...], (((1,), (1,)), ((), ())),
                      preferred_element_type=jnp.float32)
  h = h + lax.dot_general(mean, wr_ref[...], (((1,), (1,)), ((), ())),
                          preferred_element_type=jnp.float32)
  h = h + b_ref[...]
  nrm = jnp.sqrt(jnp.sum(h * h, axis=1, keepdims=True))
  h = h / jnp.maximum(nrm, 1e-12)
  o_ref[...] = jnp.maximum(h, 0.0)


def _tc_final_body(x_ref, a_ref, c_ref, wl_ref, wr_ref, b_ref, wp_ref, bp_ref,
                   o_ref):
  a = a_ref[0] + a_ref[1]
  cnt = c_ref[0, :, 0:1] + c_ref[1, :, 0:1]
  mean = a / jnp.maximum(cnt, 1.0)
  h = lax.dot_general(x_ref[...], wl_ref[...], (((1,), (1,)), ((), ())),
                      preferred_element_type=jnp.float32)
  h = h + lax.dot_general(mean, wr_ref[...], (((1,), (1,)), ((), ())),
                          preferred_element_type=jnp.float32)
  h = h + b_ref[...]
  nrm = jnp.sqrt(jnp.sum(h * h, axis=1, keepdims=True))
  h = h / jnp.maximum(nrm, 1e-12)
  h = jnp.maximum(h, 0.0)
  o = lax.dot_general(h, wp_ref[...], (((1,), (1,)), ((), ())),
                      preferred_element_type=jnp.float32)
  o = o + bp_ref[...]
  nrm = jnp.sqrt(jnp.sum(o * o, axis=1, keepdims=True))
  o_ref[...] = o / jnp.maximum(nrm, 1e-12)


def _row_specs():
  xs = pl.BlockSpec((_TB, D), lambda i: (i, 0))
  accs = pl.BlockSpec((NC, _TB, D), lambda i: (0, i, 0))
  cnts = pl.BlockSpec((NC, _TB, 16), lambda i: (0, i, 0))
  w = pl.BlockSpec((D, D), lambda i: (0, 0))
  b = pl.BlockSpec((1, D), lambda i: (0, 0))
  return xs, accs, cnts, w, b


def _tc_layer(x, acc, cnt, wl, wr, bsum):
  xs, accs, cnts, w, b = _row_specs()
  return pl.pallas_call(
      _tc_layer_body,
      grid=(N // _TB,),
      in_specs=[xs, accs, cnts, w, w, b],
      out_specs=xs,
      out_shape=jax.ShapeDtypeStruct((N, D), jnp.float32),
  )(x, acc, cnt, wl, wr, bsum)


def _tc_final(x, acc, cnt, wl, wr, bsum, wp, bp):
  xs, accs, cnts, w, b = _row_specs()
  return pl.pallas_call(
      _tc_final_body,
      grid=(N // _TB,),
      in_specs=[xs, accs, cnts, w, w, b, w, b],
      out_specs=xs,
      out_shape=jax.ShapeDtypeStruct((N, D), jnp.float32),
  )(x, acc, cnt, wl, wr, bsum, wp, bp)


def kernel(x, edge_index, score, Wl0, bl0, Wr0, br0, Wl1, bl1, Wr1, br1,
           Wp, bp):
  src = edge_index[0]
  dst = edge_index[1]
  pad = E_PAD - E
  srcp = jnp.concatenate([src, jnp.zeros((pad,), jnp.int32)])
  dstp = jnp.concatenate([dst, jnp.full((pad,), N, jnp.int32)])
  scorep = jnp.concatenate([score, jnp.zeros((pad,), jnp.float32)])

  zrows = jnp.zeros((RPT, D), jnp.float32)
  zcnt = jnp.zeros((RPT, 16), jnp.float32)
  ones = jnp.ones((CH, 16), jnp.float32)

  def _agg_xla(feats):
    msg = feats[src] * score[:, None]
    summed = jax.ops.segment_sum(msg, dst, num_segments=N)
    c = jax.ops.segment_sum(jnp.ones((E,), jnp.float32), dst, num_segments=N)
    acc = jnp.zeros((NC, NPAD, D), jnp.float32).at[0, :N].set(summed)
    cntp = jnp.zeros((NC, NPAD, 16), jnp.float32).at[0, :N, 0].set(c)
    return acc, cntp

  acc1, cnt = _agg_xla(x)
  h1 = _tc_layer(x, acc1, cnt, Wl0, Wr0, (bl0 + br0)[None, :])
  acc2, _ = _agg_xla(h1)
  return _tc_final(h1, acc2, cnt, Wl1, Wr1, (bl1 + br1)[None, :], Wp,
                   bp[None, :])


# speedup vs baseline: 3.3247x; 3.2743x over previous
import jax
import jax.numpy as jnp
from jax import lax
from jax.experimental import pallas as pl
from jax.experimental.pallas import tpu as pltpu
from jax.experimental.pallas import tpu_sc as plsc

N = 10000
D = 128
E = 320000

NC = 2
NS = 16
NW = NC * NS

CH = 128
NCHUNK = -(-E // (NW * CH))
EPT = NCHUNK * CH
E_PAD = EPT * NW

NPAD = 10112
RPT = NPAD // NS


def _make_sc_aggregate(with_cnt: bool):
  mesh = plsc.VectorSubcoreMesh(core_axis_name="c", subcore_axis_name="s")

  out_type = [jax.ShapeDtypeStruct((NC, NPAD, D), jnp.float32)]
  if with_cnt:
    out_type.append(jax.ShapeDtypeStruct((NC, NPAD, D), jnp.float32))

  scratch = [
      pltpu.VMEM_SHARED((NPAD, D), jnp.float32),
      pltpu.VMEM((CH,), jnp.int32),
      pltpu.VMEM((CH,), jnp.int32),
      pltpu.VMEM((CH,), jnp.float32),
      pltpu.VMEM((CH, D), jnp.float32),
      pltpu.SemaphoreType.DMA,
  ]

  def body(feats, srcp, dstp, scorep, zrows, ones, *rest):
    if with_cnt:
      acc_out, cnt_out, acc_s, src_v, dst_v, sc_v, rows_v, sem = rest
    else:
      acc_out, acc_s, src_v, dst_v, sc_v, rows_v, sem = rest
    c = lax.axis_index("c")
    s = lax.axis_index("s")
    wid = s * NC + c
    base0 = wid * EPT

    pltpu.sync_copy(zrows, acc_s.at[pl.ds(s * RPT, RPT)])

    if with_cnt:
      pltpu.sync_copy(ones, rows_v)
      plsc.subcore_barrier()

      def cnt_chunk(i, carry):
        pltpu.sync_copy(dstp.at[pl.ds(base0 + i * CH, CH)], dst_v)
        pltpu.sync_copy(rows_v, acc_s.at[dst_v], add=True)
        return carry

      lax.fori_loop(0, NCHUNK, cnt_chunk, 0)
      plsc.subcore_barrier()
      pltpu.sync_copy(acc_s.at[pl.ds(s * RPT, RPT)],
                      cnt_out.at[c, pl.ds(s * RPT, RPT)])
      pltpu.sync_copy(zrows, acc_s.at[pl.ds(s * RPT, RPT)])

    plsc.subcore_barrier()

    def chunk(i, carry):
      b = base0 + i * CH
      pltpu.sync_copy(srcp.at[pl.ds(b, CH)], src_v)
      pltpu.sync_copy(dstp.at[pl.ds(b, CH)], dst_v)
      pltpu.sync_copy(scorep.at[pl.ds(b, CH)], sc_v)
      pltpu.async_copy(feats.at[src_v], rows_v, sem).wait()

      for g in range(CH // 16):
        sv16 = sc_v[pl.ds(g * 16, 16)]
        for j in range(16):
          k = g * 16 + j
          sv = jnp.full((16,), sv16[j], jnp.float32)
          for f in range(D // 16):
            rows_v[k, pl.ds(f * 16, 16)] = rows_v[k, pl.ds(f * 16, 16)] * sv

      pltpu.sync_copy(rows_v, acc_s.at[dst_v], add=True)
      return carry

    lax.fori_loop(0, NCHUNK, chunk, 0)
    plsc.subcore_barrier()

    pltpu.sync_copy(acc_s.at[pl.ds(s * RPT, RPT)],
                    acc_out.at[c, pl.ds(s * RPT, RPT)])

  return pl.kernel(body, out_type=tuple(out_type), mesh=mesh,
                   scratch_types=scratch)


_sc_agg_cnt = _make_sc_aggregate(True)
_sc_agg = _make_sc_aggregate(False)

_TB = 1000


def _tc_layer_body(x_ref, a_ref, c_ref, wl_ref, wr_ref, b_ref, o_ref):
  a = a_ref[0] + a_ref[1]
  cnt = c_ref[0, :, 0:1] + c_ref[1, :, 0:1]
  mean = a / jnp.maximum(cnt, 1.0)
  h = lax.dot_general(x_ref[...], wl_ref[...], (((1,), (1,)), ((), ())),
                      preferred_element_type=jnp.float32)
  h = h + lax.dot_general(mean, wr_ref[...], (((1,), (1,)), ((), ())),
                          preferred_element_type=jnp.float32)
  h = h + b_ref[...]
  nrm = jnp.sqrt(jnp.sum(h * h, axis=1, keepdims=True))
  h = h / jnp.maximum(nrm, 1e-12)
  o_ref[...] = jnp.maximum(h, 0.0)


def _tc_final_body(x_ref, a_ref, c_ref, wl_ref, wr_ref, b_ref, wp_ref, bp_ref,
                   o_ref):
  a = a_ref[0] + a_ref[1]
  cnt = c_ref[0, :, 0:1] + c_ref[1, :, 0:1]
  mean = a / jnp.maximum(cnt, 1.0)
  h = lax.dot_general(x_ref[...], wl_ref[...], (((1,), (1,)), ((), ())),
                      preferred_element_type=jnp.float32)
  h = h + lax.dot_general(mean, wr_ref[...], (((1,), (1,)), ((), ())),
                          preferred_element_type=jnp.float32)
  h = h + b_ref[...]
  nrm = jnp.sqrt(jnp.sum(h * h, axis=1, keepdims=True))
  h = h / jnp.maximum(nrm, 1e-12)
  h = jnp.maximum(h, 0.0)
  o = lax.dot_general(h, wp_ref[...], (((1,), (1,)), ((), ())),
                      preferred_element_type=jnp.float32)
  o = o + bp_ref[...]
  nrm = jnp.sqrt(jnp.sum(o * o, axis=1, keepdims=True))
  o_ref[...] = o / jnp.maximum(nrm, 1e-12)


def _row_specs():
  xs = pl.BlockSpec((_TB, D), lambda i: (i, 0))
  accs = pl.BlockSpec((NC, _TB, D), lambda i: (0, i, 0))
  w = pl.BlockSpec((D, D), lambda i: (0, 0))
  b = pl.BlockSpec((1, D), lambda i: (0, 0))
  return xs, accs, w, b


def _tc_layer(x, acc, cnt, wl, wr, bsum):
  xs, accs, w, b = _row_specs()
  return pl.pallas_call(
      _tc_layer_body,
      grid=(N // _TB,),
      in_specs=[xs, accs, accs, w, w, b],
      out_specs=xs,
      out_shape=jax.ShapeDtypeStruct((N, D), jnp.float32),
  )(x, acc, cnt, wl, wr, bsum)


def _tc_final(x, acc, cnt, wl, wr, bsum, wp, bp):
  xs, accs, w, b = _row_specs()
  return pl.pallas_call(
      _tc_final_body,
      grid=(N // _TB,),
      in_specs=[xs, accs, accs, w, w, b, w, b],
      out_specs=xs,
      out_shape=jax.ShapeDtypeStruct((N, D), jnp.float32),
  )(x, acc, cnt, wl, wr, bsum, wp, bp)


def kernel(x, edge_index, score, Wl0, bl0, Wr0, br0, Wl1, bl1, Wr1, br1,
           Wp, bp):
  src = edge_index[0]
  dst = edge_index[1]
  pad = E_PAD - E
  srcp = jnp.concatenate([src, jnp.zeros((pad,), jnp.int32)])
  dstp = jnp.concatenate([dst, jnp.full((pad,), N, jnp.int32)])
  scorep = jnp.concatenate([score, jnp.zeros((pad,), jnp.float32)])

  zrows = jnp.zeros((RPT, D), jnp.float32)
  ones = jnp.ones((CH, D), jnp.float32)

  acc1, cnt = _sc_agg_cnt(x, srcp, dstp, scorep, zrows, ones)
  h1 = _tc_layer(x, acc1, cnt, Wl0, Wr0, (bl0 + br0)[None, :])
  (acc2,) = _sc_agg(h1, srcp, dstp, scorep, zrows, ones)
  return _tc_final(h1, acc2, cnt, Wl1, Wr1, (bl1 + br1)[None, :], Wp,
                   bp[None, :])
